# Initial kernel scaffold; baseline (speedup 1.0000x reference)
import functools

import jax
import jax.numpy as jnp
from jax import lax
from jax.experimental import pallas as pl
from jax.experimental.pallas import tpu as pltpu
from jax.experimental.pallas import tpu_sc as plsc

N_NODES = 10000
N_EDGES = 320000
F_IN = 128
HIDDEN = 64
N_GRAPHS = 512

NC = 2
NS = 16
NW = NC * NS
CH = 128
NJ = 79
E_PAD = NW * NJ * CH
N_PAD = 10240
RPT = N_PAD // NS

_MESH = dict(core_axis_name="c", subcore_axis_name="s")


def _deg_body(row3, degp, rowv, ones_v, zb, acc):
    c = lax.axis_index("c")
    s = lax.axis_index("s")
    wid = s * NC + c
    for i in range(CH // 16):
        ones_v[pl.ds(i * 16, 16)] = jnp.ones((16,), jnp.float32)
    for i in range(RPT // 16):
        zb[pl.ds(i * 16, 16)] = jnp.zeros((16,), jnp.float32)
    pltpu.sync_copy(zb, acc.at[pl.ds(s * RPT, RPT)])
    plsc.subcore_barrier()
    pltpu.sync_copy(row3.at[wid], rowv)

    def body(j, carry):
        pltpu.sync_copy(ones_v, acc.at[rowv.at[j]], add=True)
        return carry

    lax.fori_loop(0, NJ, body, 0)
    plsc.subcore_barrier()
    pltpu.sync_copy(acc.at[pl.ds(s * RPT, RPT)],
                    degp.at[c, pl.ds(s * RPT, RPT)])


_deg_kernel = pl.kernel(
    _deg_body,
    out_type=jax.ShapeDtypeStruct((NC, N_PAD), jnp.float32),
    mesh=plsc.VectorSubcoreMesh(**_MESH),
    scratch_types=[
        pltpu.VMEM((NJ, CH), jnp.int32),
        pltpu.VMEM((CH,), jnp.float32),
        pltpu.VMEM((RPT,), jnp.float32),
        pltpu.VMEM_SHARED((N_PAD,), jnp.float32),
    ],
)


def _scat_body(yt, row3, col3, p, rowv, colv, rows_v, zbuf, acc, sem):
    c = lax.axis_index("c")
    s = lax.axis_index("s")
    wid = s * NC + c

    def zbody(i, carry):
        for k in range(HIDDEN // 16):
            zbuf[i, pl.ds(k * 16, 16)] = jnp.zeros((16,), jnp.float32)
        return carry

    lax.fori_loop(0, RPT, zbody, 0)
    pltpu.sync_copy(zbuf, acc.at[pl.ds(s * RPT, RPT)])
    plsc.subcore_barrier()
    pltpu.sync_copy(row3.at[wid], rowv)
    pltpu.sync_copy(col3.at[wid], colv)

    def body(j, carry):
        pltpu.async_copy(yt.at[colv.at[j]], rows_v, sem).wait()
        pltpu.sync_copy(rows_v, acc.at[rowv.at[j]], add=True)
        return carry

    lax.fori_loop(0, NJ, body, 0)
    plsc.subcore_barrier()
    pltpu.sync_copy(acc.at[pl.ds(s * RPT, RPT)],
                    p.at[c, pl.ds(s * RPT, RPT)])


_scat_kernel = pl.kernel(
    _scat_body,
    out_type=jax.ShapeDtypeStruct((NC, N_PAD, HIDDEN), jnp.float32),
    mesh=plsc.VectorSubcoreMesh(**_MESH),
    scratch_types=[
        pltpu.VMEM((NJ, CH), jnp.int32),
        pltpu.VMEM((NJ, CH), jnp.int32),
        pltpu.VMEM((CH, HIDDEN), jnp.float32),
        pltpu.VMEM((RPT, HIDDEN), jnp.float32),
        pltpu.VMEM_SHARED((N_PAD, HIDDEN), jnp.float32),
        pltpu.SemaphoreType.DMA,
    ],
)


def _tc1_body(x_ref, degT_ref, lmax_ref, batchc_ref, W_ref, b_ref,
              yt_ref, base_ref, c_ref, dinv_ref, diag_ref):
    deg = degT_ref[:N_NODES, 0:1] + degT_ref[:N_NODES, 1:2]
    dinv = jnp.where(deg > 0, lax.rsqrt(jnp.maximum(deg, 1.0)), 0.0)
    iota = lax.broadcasted_iota(jnp.int32, (N_NODES, N_GRAPHS), 1)
    oh = (batchc_ref[...] == iota).astype(jnp.float32)
    lam = jnp.dot(oh, lmax_ref[...], preferred_element_type=jnp.float32)
    ilam = 2.0 / lam
    cvec = ilam * dinv
    diag = ilam - 1.0
    x = x_ref[...]
    W = W_ref[...]
    y = jnp.dot(x, W[1], preferred_element_type=jnp.float32)
    yt_ref[...] = dinv * y
    base_ref[...] = (jnp.dot(x, W[0], preferred_element_type=jnp.float32)
                     + diag * y + b_ref[...])
    c_ref[...] = cvec
    dinv_ref[...] = dinv
    diag_ref[...] = diag


_tc1_kernel = pl.pallas_call(
    _tc1_body,
    out_shape=(
        jax.ShapeDtypeStruct((N_NODES, HIDDEN), jnp.float32),
        jax.ShapeDtypeStruct((N_NODES, HIDDEN), jnp.float32),
        jax.ShapeDtypeStruct((N_NODES, 1), jnp.float32),
        jax.ShapeDtypeStruct((N_NODES, 1), jnp.float32),
        jax.ShapeDtypeStruct((N_NODES, 1), jnp.float32),
    ),
)


def _tcl_body(base_ref, p_ref, c_ref, dinv_ref, diag_ref, W_ref, b_ref,
              yt_ref, out_ref):
    ptot = p_ref[0, :N_NODES, :] + p_ref[1, :N_NODES, :]
    z = jnp.maximum(base_ref[...] - c_ref[...] * ptot, 0.0)
    W = W_ref[...]
    y = jnp.dot(z, W[1], preferred_element_type=jnp.float32)
    yt_ref[...] = dinv_ref[...] * y
    out_ref[...] = (jnp.dot(z, W[0], preferred_element_type=jnp.float32)
                    + diag_ref[...] * y + b_ref[...])


_tcl_kernel = pl.pallas_call(
    _tcl_body,
    out_shape=(
        jax.ShapeDtypeStruct((N_NODES, HIDDEN), jnp.float32),
        jax.ShapeDtypeStruct((N_NODES, HIDDEN), jnp.float32),
    ),
)


def _fin_body(base_ref, p_ref, c_ref, batchr_ref,
              fc1w_ref, fc1b_ref, fc2w_ref, fc2b_ref, out_ref):
    ptot = p_ref[0, :N_NODES, :] + p_ref[1, :N_NODES, :]
    h = jnp.maximum(base_ref[...] - c_ref[...] * ptot, 0.0)
    iota = lax.broadcasted_iota(jnp.int32, (N_GRAPHS, N_NODES), 0)
    oh = (batchr_ref[...] == iota).astype(jnp.float32)
    g = jnp.dot(oh, h, preferred_element_type=jnp.float32)
    g1 = jnp.maximum(
        jnp.dot(g, fc1w_ref[...], preferred_element_type=jnp.float32)
        + fc1b_ref[...], 0.0)
    out_ref[...] = (jnp.dot(g1, fc2w_ref[...], preferred_element_type=jnp.float32)
                    + fc2b_ref[...])


_fin_kernel = pl.pallas_call(
    _fin_body,
    out_shape=jax.ShapeDtypeStruct((N_GRAPHS, 1), jnp.float32),
)


def kernel(x, edge_index, lmax, batch,
           W1, b1, W2, b2, W3, b3, W4, b4,
           fc1_w, fc1_b, fc2_w, fc2_b):
    row = edge_index[0]
    col = edge_index[1]
    npad = E_PAD - N_EDGES
    pr = N_NODES + (jnp.arange(npad, dtype=jnp.int32) % (N_PAD - N_NODES))
    pc = jnp.arange(npad, dtype=jnp.int32) % N_NODES
    row3 = jnp.concatenate([row, pr]).reshape(NW, NJ, CH)
    col3 = jnp.concatenate([col, pc]).reshape(NW, NJ, CH)

    degp = _deg_kernel(row3)
    degT = degp.T

    yt, base, cvec, dinv, diag = _tc1_kernel(
        x, degT, lmax.reshape(N_GRAPHS, 1), batch.reshape(N_NODES, 1),
        W1, b1.reshape(1, HIDDEN))
    for W, b in ((W2, b2), (W3, b3), (W4, b4)):
        p = _scat_kernel(yt, row3, col3)
        yt, base = _tcl_kernel(base, p, cvec, dinv, diag, W,
                               b.reshape(1, HIDDEN))
    p = _scat_kernel(yt, row3, col3)
    return _fin_kernel(base, p, cvec, batch.reshape(1, N_NODES),
                       fc1_w, fc1_b.reshape(1, 32),
                       fc2_w, fc2_b.reshape(1, 1))


# speedup vs baseline: 22.6482x; 22.6482x over previous
import functools

import jax
import jax.numpy as jnp
from jax import lax
from jax.experimental import pallas as pl
from jax.experimental.pallas import tpu as pltpu
from jax.experimental.pallas import tpu_sc as plsc

N_NODES = 10000
N_EDGES = 320000
F_IN = 128
HIDDEN = 64
N_GRAPHS = 512

NC = 2
NS = 16
NW = NC * NS
CH = 128
NJ = 79
E_PAD = NW * NJ * CH
N_PAD = 10240
RPT = N_PAD // NS

_MESH = dict(core_axis_name="c", subcore_axis_name="s",
             num_cores=NC, num_subcores=NS)


def _deg_body(row3, degp, rowv, ones_v, zb, acc):
    c = lax.axis_index("c")
    s = lax.axis_index("s")
    wid = s * NC + c
    for i in range(CH // 16):
        ones_v[pl.ds(i * 16, 16)] = jnp.ones((16,), jnp.float32)
    for i in range(RPT // 16):
        zb[pl.ds(i * 16, 16)] = jnp.zeros((16,), jnp.float32)
    pltpu.sync_copy(zb, acc.at[pl.ds(s * RPT, RPT)])
    plsc.subcore_barrier()
    pltpu.sync_copy(row3.at[wid], rowv)

    def body(j, carry):
        pltpu.sync_copy(ones_v, acc.at[rowv.at[j]], add=True)
        return carry

    lax.fori_loop(0, NJ, body, 0)
    plsc.subcore_barrier()
    pltpu.sync_copy(acc.at[pl.ds(s * RPT, RPT)],
                    degp.at[c, pl.ds(s * RPT, RPT)])


@functools.cache
def _deg_kernel():
    return pl.kernel(
        _deg_body,
        out_type=jax.ShapeDtypeStruct((NC, N_PAD), jnp.float32),
        mesh=plsc.VectorSubcoreMesh(**_MESH),
        scratch_types=[
            pltpu.VMEM((NJ, CH), jnp.int32),
            pltpu.VMEM((CH,), jnp.float32),
            pltpu.VMEM((RPT,), jnp.float32),
            pltpu.VMEM_SHARED((N_PAD,), jnp.float32),
        ],
    )


def _scat_body(yt, row3, col3, p, rowv, colv, rows_v, zbuf, acc, sem):
    c = lax.axis_index("c")
    s = lax.axis_index("s")
    wid = s * NC + c

    def zbody(i, carry):
        for k in range(HIDDEN // 16):
            zbuf[i, pl.ds(k * 16, 16)] = jnp.zeros((16,), jnp.float32)
        return carry

    lax.fori_loop(0, RPT, zbody, 0)
    pltpu.sync_copy(zbuf, acc.at[pl.ds(s * RPT, RPT)])
    plsc.subcore_barrier()
    pltpu.sync_copy(row3.at[wid], rowv)
    pltpu.sync_copy(col3.at[wid], colv)

    def body(j, carry):
        pltpu.async_copy(yt.at[colv.at[j]], rows_v, sem).wait()
        pltpu.sync_copy(rows_v, acc.at[rowv.at[j]], add=True)
        return carry

    lax.fori_loop(0, NJ, body, 0)
    plsc.subcore_barrier()
    pltpu.sync_copy(acc.at[pl.ds(s * RPT, RPT)],
                    p.at[c, pl.ds(s * RPT, RPT)])


@functools.cache
def _scat_kernel():
    return pl.kernel(
        _scat_body,
        out_type=jax.ShapeDtypeStruct((NC, N_PAD, HIDDEN), jnp.float32),
        mesh=plsc.VectorSubcoreMesh(**_MESH),
        scratch_types=[
            pltpu.VMEM((NJ, CH), jnp.int32),
            pltpu.VMEM((NJ, CH), jnp.int32),
            pltpu.VMEM((CH, HIDDEN), jnp.float32),
            pltpu.VMEM((RPT, HIDDEN), jnp.float32),
            pltpu.VMEM_SHARED((N_PAD, HIDDEN), jnp.float32),
            pltpu.SemaphoreType.DMA,
        ],
        compiler_params=pltpu.CompilerParams(use_tc_tiling_on_sc=False),
    )


def _tc1_body(x_ref, degT_ref, lmax_ref, batchc_ref, W_ref, b_ref,
              yt_ref, base_ref, c_ref, dinv_ref, diag_ref):
    deg = degT_ref[:N_NODES, 0:1] + degT_ref[:N_NODES, 1:2]
    dinv = jnp.where(deg > 0, lax.rsqrt(jnp.maximum(deg, 1.0)), 0.0)
    iota = lax.broadcasted_iota(jnp.int32, (N_NODES, N_GRAPHS), 1)
    oh = (batchc_ref[...] == iota).astype(jnp.float32)
    lam = jnp.dot(oh, lmax_ref[...], preferred_element_type=jnp.float32)
    ilam = 2.0 / lam
    cvec = ilam * dinv
    diag = ilam - 1.0
    x = x_ref[...]
    W = W_ref[...]
    y = jnp.dot(x, W[1], preferred_element_type=jnp.float32)
    yt_ref[...] = dinv * y
    base_ref[...] = (jnp.dot(x, W[0], preferred_element_type=jnp.float32)
                     + diag * y + b_ref[...])
    c_ref[...] = cvec
    dinv_ref[...] = dinv
    diag_ref[...] = diag


_tc1_kernel = pl.pallas_call(
    _tc1_body,
    out_shape=(
        jax.ShapeDtypeStruct((N_NODES, HIDDEN), jnp.float32),
        jax.ShapeDtypeStruct((N_NODES, HIDDEN), jnp.float32),
        jax.ShapeDtypeStruct((N_NODES, 1), jnp.float32),
        jax.ShapeDtypeStruct((N_NODES, 1), jnp.float32),
        jax.ShapeDtypeStruct((N_NODES, 1), jnp.float32),
    ),
)


def _tcl_body(base_ref, p_ref, c_ref, dinv_ref, diag_ref, W_ref, b_ref,
              yt_ref, out_ref):
    ptot = p_ref[0, :N_NODES, :] + p_ref[1, :N_NODES, :]
    z = jnp.maximum(base_ref[...] - c_ref[...] * ptot, 0.0)
    W = W_ref[...]
    y = jnp.dot(z, W[1], preferred_element_type=jnp.float32)
    yt_ref[...] = dinv_ref[...] * y
    out_ref[...] = (jnp.dot(z, W[0], preferred_element_type=jnp.float32)
                    + diag_ref[...] * y + b_ref[...])


_tcl_kernel = pl.pallas_call(
    _tcl_body,
    out_shape=(
        jax.ShapeDtypeStruct((N_NODES, HIDDEN), jnp.float32),
        jax.ShapeDtypeStruct((N_NODES, HIDDEN), jnp.float32),
    ),
)


def _fin_body(base_ref, p_ref, c_ref, batchr_ref,
              fc1w_ref, fc1b_ref, fc2w_ref, fc2b_ref, out_ref):
    ptot = p_ref[0, :N_NODES, :] + p_ref[1, :N_NODES, :]
    h = jnp.maximum(base_ref[...] - c_ref[...] * ptot, 0.0)
    iota = lax.broadcasted_iota(jnp.int32, (N_GRAPHS, N_NODES), 0)
    oh = (batchr_ref[...] == iota).astype(jnp.float32)
    g = jnp.dot(oh, h, preferred_element_type=jnp.float32)
    g1 = jnp.maximum(
        jnp.dot(g, fc1w_ref[...], preferred_element_type=jnp.float32)
        + fc1b_ref[...], 0.0)
    out_ref[...] = (jnp.dot(g1, fc2w_ref[...], preferred_element_type=jnp.float32)
                    + fc2b_ref[...])


_fin_kernel = pl.pallas_call(
    _fin_body,
    out_shape=jax.ShapeDtypeStruct((N_GRAPHS, 1), jnp.float32),
)


def kernel(x, edge_index, lmax, batch,
           W1, b1, W2, b2, W3, b3, W4, b4,
           fc1_w, fc1_b, fc2_w, fc2_b):
    row = edge_index[0]
    col = edge_index[1]
    npad = E_PAD - N_EDGES
    pr = N_NODES + (jnp.arange(npad, dtype=jnp.int32) % (N_PAD - N_NODES))
    pc = jnp.arange(npad, dtype=jnp.int32) % N_NODES
    row3 = jnp.concatenate([row, pr]).reshape(NW, NJ, CH)
    col3 = jnp.concatenate([col, pc]).reshape(NW, NJ, CH)

    degp = _deg_kernel()(row3)
    degT = degp.T

    yt, base, cvec, dinv, diag = _tc1_kernel(
        x, degT, lmax.reshape(N_GRAPHS, 1), batch.reshape(N_NODES, 1),
        W1, b1.reshape(1, HIDDEN))
    for W, b in ((W2, b2), (W3, b3), (W4, b4)):
        p = _scat_kernel()(yt, row3, col3)
        yt, base = _tcl_kernel(base, p, cvec, dinv, diag, W,
                               b.reshape(1, HIDDEN))
    p = _scat_kernel()(yt, row3, col3)
    return _fin_kernel(base, p, cvec, batch.reshape(1, N_NODES),
                       fc1_w, fc1_b.reshape(1, 32),
                       fc2_w, fc2_b.reshape(1, 1))


# speedup vs baseline: 31.6776x; 1.3987x over previous
import functools

import jax
import jax.numpy as jnp
from jax import lax
from jax.experimental import pallas as pl
from jax.experimental.pallas import tpu as pltpu
from jax.experimental.pallas import tpu_sc as plsc

N_NODES = 10000
N_EDGES = 320000
F_IN = 128
HIDDEN = 64
N_GRAPHS = 512

NC = 2
NS = 16
NW = NC * NS
CH = 128
NJ = 80
NB = 4
E_PAD = NW * NJ * CH
NJ2 = E_PAD // (NS * CH)
NG2 = NJ2 // NB
HH = HIDDEN // 2
N_PAD = 10240
RPT = N_PAD // NS

_MESH = dict(core_axis_name="c", subcore_axis_name="s",
             num_cores=NC, num_subcores=NS)


def _deg_body(row3, degp, rowv, ones_v, zb, acc):
    c = lax.axis_index("c")
    s = lax.axis_index("s")
    wid = s * NC + c
    for i in range(CH // 16):
        ones_v[pl.ds(i * 16, 16)] = jnp.ones((16,), jnp.float32)
    for i in range(RPT // 16):
        zb[pl.ds(i * 16, 16)] = jnp.zeros((16,), jnp.float32)
    pltpu.sync_copy(zb, acc.at[pl.ds(s * RPT, RPT)])
    plsc.subcore_barrier()
    pltpu.sync_copy(row3.at[wid], rowv)

    def body(j, carry):
        pltpu.sync_copy(ones_v, acc.at[rowv.at[j]], add=True)
        return carry

    lax.fori_loop(0, NJ, body, 0)
    plsc.subcore_barrier()
    pltpu.sync_copy(acc.at[pl.ds(s * RPT, RPT)],
                    degp.at[c, pl.ds(s * RPT, RPT)])


@functools.cache
def _deg_kernel():
    return pl.kernel(
        _deg_body,
        out_type=jax.ShapeDtypeStruct((NC, N_PAD), jnp.float32),
        mesh=plsc.VectorSubcoreMesh(**_MESH),
        scratch_types=[
            pltpu.VMEM((NJ, CH), jnp.int32),
            pltpu.VMEM((CH,), jnp.float32),
            pltpu.VMEM((RPT,), jnp.float32),
            pltpu.VMEM_SHARED((N_PAD,), jnp.float32),
        ],
    )


def _scat_body(yt2, row4, col4, p, rowv, colv,
               b0, b1, b2, b3, zbuf, acc,
               sg0, sg1, sg2, sg3, ss0, ss1, ss2, ss3):
    bufs = (b0, b1, b2, b3)
    sgs = (sg0, sg1, sg2, sg3)
    sss = (ss0, ss1, ss2, ss3)
    c = lax.axis_index("c")
    s = lax.axis_index("s")
    wid = c * NS + s

    def zbody(i, carry):
        for k in range(HH // 16):
            zbuf[i, pl.ds(k * 16, 16)] = jnp.zeros((16,), jnp.float32)
        return carry

    lax.fori_loop(0, RPT, zbody, 0)
    pltpu.sync_copy(row4.at[wid], rowv)
    pltpu.sync_copy(col4.at[wid], colv)
    pltpu.sync_copy(zbuf, acc.at[pl.ds(s * RPT, RPT)])
    for b in range(NB):
        pltpu.async_copy(yt2.at[colv.at[b]], bufs[b], sgs[b])
    plsc.subcore_barrier()

    def body(g, carry):
        for b in range(NB):
            j = g * NB + b
            pltpu.make_async_copy(yt2.at[colv.at[j]], bufs[b], sgs[b]).wait()
            pltpu.async_copy(bufs[b], acc.at[rowv.at[j]], sss[b], add=True)
        for b in range(NB):
            j = g * NB + b
            pltpu.make_async_copy(bufs[b], acc.at[rowv.at[j]], sss[b]).wait()

            @pl.when(g < NG2 - 1)
            def _():
                pltpu.async_copy(yt2.at[colv.at[(g + 1) * NB + b]],
                                 bufs[b], sgs[b])
        return carry

    lax.fori_loop(0, NG2, body, 0)
    plsc.subcore_barrier()
    pltpu.sync_copy(acc.at[pl.ds(s * RPT, RPT)],
                    p.at[c, pl.ds(s * RPT, RPT)])


@functools.cache
def _scat_kernel():
    return pl.kernel(
        _scat_body,
        out_type=jax.ShapeDtypeStruct((NC, N_PAD, HH), jnp.float32),
        mesh=plsc.VectorSubcoreMesh(**_MESH),
        scratch_types=[
            pltpu.VMEM((NJ2, CH), jnp.int32),
            pltpu.VMEM((NJ2, CH), jnp.int32),
            pltpu.VMEM((CH, HH), jnp.float32),
            pltpu.VMEM((CH, HH), jnp.float32),
            pltpu.VMEM((CH, HH), jnp.float32),
            pltpu.VMEM((CH, HH), jnp.float32),
            pltpu.VMEM((RPT, HH), jnp.float32),
            pltpu.VMEM_SHARED((N_PAD, HH), jnp.float32),
            pltpu.SemaphoreType.DMA,
            pltpu.SemaphoreType.DMA,
            pltpu.SemaphoreType.DMA,
            pltpu.SemaphoreType.DMA,
            pltpu.SemaphoreType.DMA,
            pltpu.SemaphoreType.DMA,
            pltpu.SemaphoreType.DMA,
            pltpu.SemaphoreType.DMA,
        ],
        compiler_params=pltpu.CompilerParams(use_tc_tiling_on_sc=False),
    )


RB = 2000
NRB = N_NODES // RB


def _tc1_body(x_ref, degT_ref, lmax_ref, batchc_ref, W_ref, b_ref,
              yt_ref, base_ref, c_ref, dinv_ref, diag_ref):
    deg = degT_ref[:, 0:1] + degT_ref[:, 1:2]
    dinv = jnp.where(deg > 0, lax.rsqrt(jnp.maximum(deg, 1.0)), 0.0)
    iota = lax.broadcasted_iota(jnp.int32, (RB, N_GRAPHS), 1)
    oh = (batchc_ref[...] == iota).astype(jnp.float32)
    lam = jnp.dot(oh, lmax_ref[...], preferred_element_type=jnp.float32)
    ilam = 2.0 / lam
    cvec = ilam * dinv
    diag = ilam - 1.0
    x = x_ref[...]
    W = W_ref[...]
    y = jnp.dot(x, W[1], preferred_element_type=jnp.float32)
    yt = dinv * y
    yt_ref[0] = yt[:, :HH]
    yt_ref[1] = yt[:, HH:]
    base_ref[...] = (jnp.dot(x, W[0], preferred_element_type=jnp.float32)
                     + diag * y + b_ref[...])
    c_ref[...] = cvec
    dinv_ref[...] = dinv
    diag_ref[...] = diag


_tc1_kernel = pl.pallas_call(
    _tc1_body,
    grid=(NRB,),
    in_specs=[
        pl.BlockSpec((RB, F_IN), lambda i: (i, 0)),
        pl.BlockSpec((RB, NC), lambda i: (i, 0)),
        pl.BlockSpec((N_GRAPHS, 1), lambda i: (0, 0)),
        pl.BlockSpec((RB, 1), lambda i: (i, 0)),
        pl.BlockSpec((2, F_IN, HIDDEN), lambda i: (0, 0, 0)),
        pl.BlockSpec((1, HIDDEN), lambda i: (0, 0)),
    ],
    out_specs=(
        pl.BlockSpec((2, RB, HH), lambda i: (0, i, 0)),
        pl.BlockSpec((RB, HIDDEN), lambda i: (i, 0)),
        pl.BlockSpec((RB, 1), lambda i: (i, 0)),
        pl.BlockSpec((RB, 1), lambda i: (i, 0)),
        pl.BlockSpec((RB, 1), lambda i: (i, 0)),
    ),
    out_shape=(
        jax.ShapeDtypeStruct((2, N_NODES, HH), jnp.float32),
        jax.ShapeDtypeStruct((N_NODES, HIDDEN), jnp.float32),
        jax.ShapeDtypeStruct((N_NODES, 1), jnp.float32),
        jax.ShapeDtypeStruct((N_NODES, 1), jnp.float32),
        jax.ShapeDtypeStruct((N_NODES, 1), jnp.float32),
    ),
)


def _tcl_body(base_ref, p_ref, c_ref, dinv_ref, diag_ref, W_ref, b_ref,
              yt_ref, out_ref):
    ptot = jnp.concatenate([p_ref[0], p_ref[1]], axis=1)
    z = jnp.maximum(base_ref[...] - c_ref[...] * ptot, 0.0)
    W = W_ref[...]
    y = jnp.dot(z, W[1], preferred_element_type=jnp.float32)
    yt = dinv_ref[...] * y
    yt_ref[0] = yt[:, :HH]
    yt_ref[1] = yt[:, HH:]
    out_ref[...] = (jnp.dot(z, W[0], preferred_element_type=jnp.float32)
                    + diag_ref[...] * y + b_ref[...])


_tcl_kernel = pl.pallas_call(
    _tcl_body,
    grid=(NRB,),
    in_specs=[
        pl.BlockSpec((RB, HIDDEN), lambda i: (i, 0)),
        pl.BlockSpec((NC, RB, HH), lambda i: (0, i, 0)),
        pl.BlockSpec((RB, 1), lambda i: (i, 0)),
        pl.BlockSpec((RB, 1), lambda i: (i, 0)),
        pl.BlockSpec((RB, 1), lambda i: (i, 0)),
        pl.BlockSpec((2, HIDDEN, HIDDEN), lambda i: (0, 0, 0)),
        pl.BlockSpec((1, HIDDEN), lambda i: (0, 0)),
    ],
    out_specs=(
        pl.BlockSpec((2, RB, HH), lambda i: (0, i, 0)),
        pl.BlockSpec((RB, HIDDEN), lambda i: (i, 0)),
    ),
    out_shape=(
        jax.ShapeDtypeStruct((2, N_NODES, HH), jnp.float32),
        jax.ShapeDtypeStruct((N_NODES, HIDDEN), jnp.float32),
    ),
)


def _fin_body(base_ref, p_ref, c_ref, batchc_ref,
              fc1w_ref, fc1b_ref, fc2w_ref, fc2b_ref, out_ref, g_ref):
    i = pl.program_id(0)
    ptot = jnp.concatenate([p_ref[0], p_ref[1]], axis=1)
    h = jnp.maximum(base_ref[...] - c_ref[...] * ptot, 0.0)
    iota = lax.broadcasted_iota(jnp.int32, (RB, N_GRAPHS), 1)
    oh = (batchc_ref[...] == iota).astype(jnp.float32)
    contrib = lax.dot_general(oh, h, (((0,), (0,)), ((), ())),
                              preferred_element_type=jnp.float32)

    @pl.when(i == 0)
    def _():
        g_ref[...] = contrib

    @pl.when(i > 0)
    def _():
        g_ref[...] += contrib

    @pl.when(i == NRB - 1)
    def _():
        g = g_ref[...]
        g1 = jnp.maximum(
            jnp.dot(g, fc1w_ref[...], preferred_element_type=jnp.float32)
            + fc1b_ref[...], 0.0)
        out_ref[...] = (
            jnp.dot(g1, fc2w_ref[...], preferred_element_type=jnp.float32)
            + fc2b_ref[...])


_fin_kernel = pl.pallas_call(
    _fin_body,
    grid=(NRB,),
    in_specs=[
        pl.BlockSpec((RB, HIDDEN), lambda i: (i, 0)),
        pl.BlockSpec((NC, RB, HH), lambda i: (0, i, 0)),
        pl.BlockSpec((RB, 1), lambda i: (i, 0)),
        pl.BlockSpec((RB, 1), lambda i: (i, 0)),
        pl.BlockSpec((HIDDEN, 32), lambda i: (0, 0)),
        pl.BlockSpec((1, 32), lambda i: (0, 0)),
        pl.BlockSpec((32, 1), lambda i: (0, 0)),
        pl.BlockSpec((1, 1), lambda i: (0, 0)),
    ],
    out_specs=pl.BlockSpec((N_GRAPHS, 1), lambda i: (0, 0)),
    out_shape=jax.ShapeDtypeStruct((N_GRAPHS, 1), jnp.float32),
    scratch_shapes=[pltpu.VMEM((N_GRAPHS, HIDDEN), jnp.float32)],
)


def kernel(x, edge_index, lmax, batch,
           W1, b1, W2, b2, W3, b3, W4, b4,
           fc1_w, fc1_b, fc2_w, fc2_b):
    row = edge_index[0]
    col = edge_index[1]
    npad = E_PAD - N_EDGES
    pr = N_NODES + (jnp.arange(npad, dtype=jnp.int32) % (N_PAD - N_NODES))
    pc = jnp.arange(npad, dtype=jnp.int32) % N_NODES
    rw = jnp.concatenate([row, pr])
    cw = jnp.concatenate([col, pc])
    row3 = rw.reshape(NW, NJ, CH)
    r16 = rw.reshape(NS, NJ2, CH)
    c16 = cw.reshape(NS, NJ2, CH)
    row4 = jnp.concatenate([r16, r16], axis=0)
    col4 = jnp.concatenate([c16, c16 + N_NODES], axis=0)

    degp = _deg_kernel()(row3)
    degT = degp.T

    yt, base, cvec, dinv, diag = _tc1_kernel(
        x, degT, lmax.reshape(N_GRAPHS, 1), batch.reshape(N_NODES, 1),
        W1, b1.reshape(1, HIDDEN))
    for W, b in ((W2, b2), (W3, b3), (W4, b4)):
        p = _scat_kernel()(yt.reshape(2 * N_NODES, HH), row4, col4)
        yt, base = _tcl_kernel(base, p, cvec, dinv, diag, W,
                               b.reshape(1, HIDDEN))
    p = _scat_kernel()(yt.reshape(2 * N_NODES, HH), row4, col4)
    return _fin_kernel(base, p, cvec, batch.reshape(N_NODES, 1),
                       fc1_w, fc1_b.reshape(1, 32),
                       fc2_w, fc2_b.reshape(1, 1))


# speedup vs baseline: 33.9122x; 1.0705x over previous
import functools

import jax
import jax.numpy as jnp
from jax import lax
from jax.experimental import pallas as pl
from jax.experimental.pallas import tpu as pltpu
from jax.experimental.pallas import tpu_sc as plsc

N_NODES = 10000
N_EDGES = 320000
F_IN = 128
HIDDEN = 64
N_GRAPHS = 512

NC = 2
NS = 16
NW = NC * NS
CH = 128
NJ = 80
NB = 8
E_PAD = NW * NJ * CH
NJ2 = E_PAD // (NS * CH)
NG2 = NJ2 // NB
HH = HIDDEN // 2
N_PAD = 10240
RPT = N_PAD // NS

_MESH = dict(core_axis_name="c", subcore_axis_name="s",
             num_cores=NC, num_subcores=NS)


def _deg_body(row3, degp, rowv, ones_v, zb, acc):
    c = lax.axis_index("c")
    s = lax.axis_index("s")
    wid = s * NC + c
    for i in range(CH // 16):
        ones_v[pl.ds(i * 16, 16)] = jnp.ones((16,), jnp.float32)
    for i in range(RPT // 16):
        zb[pl.ds(i * 16, 16)] = jnp.zeros((16,), jnp.float32)
    pltpu.sync_copy(zb, acc.at[pl.ds(s * RPT, RPT)])
    plsc.subcore_barrier()
    pltpu.sync_copy(row3.at[wid], rowv)

    def body(j, carry):
        pltpu.sync_copy(ones_v, acc.at[rowv.at[j]], add=True)
        return carry

    lax.fori_loop(0, NJ, body, 0)
    plsc.subcore_barrier()
    pltpu.sync_copy(acc.at[pl.ds(s * RPT, RPT)],
                    degp.at[c, pl.ds(s * RPT, RPT)])


@functools.cache
def _deg_kernel():
    return pl.kernel(
        _deg_body,
        out_type=jax.ShapeDtypeStruct((NC, N_PAD), jnp.float32),
        mesh=plsc.VectorSubcoreMesh(**_MESH),
        scratch_types=[
            pltpu.VMEM((NJ, CH), jnp.int32),
            pltpu.VMEM((CH,), jnp.float32),
            pltpu.VMEM((RPT,), jnp.float32),
            pltpu.VMEM_SHARED((N_PAD,), jnp.float32),
        ],
    )


def _scat_body(yt2, row4, col4, p, rowv, colv, *bufs_and_sems):
    bufs = bufs_and_sems[:NB]
    zbuf, acc = bufs_and_sems[NB], bufs_and_sems[NB + 1]
    sgs = bufs_and_sems[NB + 2:2 * NB + 2]
    sss = bufs_and_sems[2 * NB + 2:]
    c = lax.axis_index("c")
    s = lax.axis_index("s")
    wid = c * NS + s

    def zbody(i, carry):
        for k in range(HH // 16):
            zbuf[i, pl.ds(k * 16, 16)] = jnp.zeros((16,), jnp.float32)
        return carry

    lax.fori_loop(0, RPT, zbody, 0)
    pltpu.sync_copy(row4.at[wid], rowv)
    pltpu.sync_copy(col4.at[wid], colv)
    pltpu.sync_copy(zbuf, acc.at[pl.ds(s * RPT, RPT)])
    for b in range(NB):
        pltpu.async_copy(yt2.at[colv.at[b]], bufs[b], sgs[b])
    plsc.subcore_barrier()

    def body(g, carry):
        for b in range(NB):
            j = g * NB + b
            pltpu.make_async_copy(yt2.at[colv.at[j]], bufs[b], sgs[b]).wait()
            pltpu.async_copy(bufs[b], acc.at[rowv.at[j]], sss[b], add=True)
        for b in range(NB):
            j = g * NB + b
            pltpu.make_async_copy(bufs[b], acc.at[rowv.at[j]], sss[b]).wait()

            @pl.when(g < NG2 - 1)
            def _():
                pltpu.async_copy(yt2.at[colv.at[(g + 1) * NB + b]],
                                 bufs[b], sgs[b])
        return carry

    lax.fori_loop(0, NG2, body, 0)
    plsc.subcore_barrier()
    pltpu.sync_copy(acc.at[pl.ds(s * RPT, RPT)],
                    p.at[c, pl.ds(s * RPT, RPT)])


@functools.cache
def _scat_kernel():
    return pl.kernel(
        _scat_body,
        out_type=jax.ShapeDtypeStruct((NC, N_PAD, HH), jnp.float32),
        mesh=plsc.VectorSubcoreMesh(**_MESH),
        scratch_types=(
            [pltpu.VMEM((NJ2, CH), jnp.int32)] * 2
            + [pltpu.VMEM((CH, HH), jnp.float32)] * NB
            + [pltpu.VMEM((RPT, HH), jnp.float32),
               pltpu.VMEM_SHARED((N_PAD, HH), jnp.float32)]
            + [pltpu.SemaphoreType.DMA] * (2 * NB)
        ),
        compiler_params=pltpu.CompilerParams(use_tc_tiling_on_sc=False),
    )


RB = 2000
NRB = N_NODES // RB


def _tc1_body(x_ref, degT_ref, lmax_ref, batchc_ref, W_ref, b_ref,
              yt_ref, base_ref, c_ref, dinv_ref, diag_ref):
    deg = degT_ref[:, 0:1] + degT_ref[:, 1:2]
    dinv = jnp.where(deg > 0, lax.rsqrt(jnp.maximum(deg, 1.0)), 0.0)
    iota = lax.broadcasted_iota(jnp.int32, (RB, N_GRAPHS), 1)
    oh = (batchc_ref[...] == iota).astype(jnp.float32)
    lam = jnp.dot(oh, lmax_ref[...], preferred_element_type=jnp.float32)
    ilam = 2.0 / lam
    cvec = ilam * dinv
    diag = ilam - 1.0
    x = x_ref[...]
    W = W_ref[...]
    y = jnp.dot(x, W[1], preferred_element_type=jnp.float32)
    yt = dinv * y
    yt_ref[0] = yt[:, :HH]
    yt_ref[1] = yt[:, HH:]
    base_ref[...] = (jnp.dot(x, W[0], preferred_element_type=jnp.float32)
                     + diag * y + b_ref[...])
    c_ref[...] = cvec
    dinv_ref[...] = dinv
    diag_ref[...] = diag


_tc1_kernel = pl.pallas_call(
    _tc1_body,
    grid=(NRB,),
    in_specs=[
        pl.BlockSpec((RB, F_IN), lambda i: (i, 0)),
        pl.BlockSpec((RB, NC), lambda i: (i, 0)),
        pl.BlockSpec((N_GRAPHS, 1), lambda i: (0, 0)),
        pl.BlockSpec((RB, 1), lambda i: (i, 0)),
        pl.BlockSpec((2, F_IN, HIDDEN), lambda i: (0, 0, 0)),
        pl.BlockSpec((1, HIDDEN), lambda i: (0, 0)),
    ],
    out_specs=(
        pl.BlockSpec((2, RB, HH), lambda i: (0, i, 0)),
        pl.BlockSpec((RB, HIDDEN), lambda i: (i, 0)),
        pl.BlockSpec((RB, 1), lambda i: (i, 0)),
        pl.BlockSpec((RB, 1), lambda i: (i, 0)),
        pl.BlockSpec((RB, 1), lambda i: (i, 0)),
    ),
    out_shape=(
        jax.ShapeDtypeStruct((2, N_NODES, HH), jnp.float32),
        jax.ShapeDtypeStruct((N_NODES, HIDDEN), jnp.float32),
        jax.ShapeDtypeStruct((N_NODES, 1), jnp.float32),
        jax.ShapeDtypeStruct((N_NODES, 1), jnp.float32),
        jax.ShapeDtypeStruct((N_NODES, 1), jnp.float32),
    ),
)


def _tcl_body(base_ref, p_ref, c_ref, dinv_ref, diag_ref, W_ref, b_ref,
              yt_ref, out_ref):
    ptot = jnp.concatenate([p_ref[0], p_ref[1]], axis=1)
    z = jnp.maximum(base_ref[...] - c_ref[...] * ptot, 0.0)
    W = W_ref[...]
    y = jnp.dot(z, W[1], preferred_element_type=jnp.float32)
    yt = dinv_ref[...] * y
    yt_ref[0] = yt[:, :HH]
    yt_ref[1] = yt[:, HH:]
    out_ref[...] = (jnp.dot(z, W[0], preferred_element_type=jnp.float32)
                    + diag_ref[...] * y + b_ref[...])


_tcl_kernel = pl.pallas_call(
    _tcl_body,
    grid=(NRB,),
    in_specs=[
        pl.BlockSpec((RB, HIDDEN), lambda i: (i, 0)),
        pl.BlockSpec((NC, RB, HH), lambda i: (0, i, 0)),
        pl.BlockSpec((RB, 1), lambda i: (i, 0)),
        pl.BlockSpec((RB, 1), lambda i: (i, 0)),
        pl.BlockSpec((RB, 1), lambda i: (i, 0)),
        pl.BlockSpec((2, HIDDEN, HIDDEN), lambda i: (0, 0, 0)),
        pl.BlockSpec((1, HIDDEN), lambda i: (0, 0)),
    ],
    out_specs=(
        pl.BlockSpec((2, RB, HH), lambda i: (0, i, 0)),
        pl.BlockSpec((RB, HIDDEN), lambda i: (i, 0)),
    ),
    out_shape=(
        jax.ShapeDtypeStruct((2, N_NODES, HH), jnp.float32),
        jax.ShapeDtypeStruct((N_NODES, HIDDEN), jnp.float32),
    ),
)


def _fin_body(base_ref, p_ref, c_ref, batchc_ref,
              fc1w_ref, fc1b_ref, fc2w_ref, fc2b_ref, out_ref, g_ref):
    i = pl.program_id(0)
    ptot = jnp.concatenate([p_ref[0], p_ref[1]], axis=1)
    h = jnp.maximum(base_ref[...] - c_ref[...] * ptot, 0.0)
    iota = lax.broadcasted_iota(jnp.int32, (RB, N_GRAPHS), 1)
    oh = (batchc_ref[...] == iota).astype(jnp.float32)
    contrib = lax.dot_general(oh, h, (((0,), (0,)), ((), ())),
                              preferred_element_type=jnp.float32)

    @pl.when(i == 0)
    def _():
        g_ref[...] = contrib

    @pl.when(i > 0)
    def _():
        g_ref[...] += contrib

    @pl.when(i == NRB - 1)
    def _():
        g = g_ref[...]
        g1 = jnp.maximum(
            jnp.dot(g, fc1w_ref[...], preferred_element_type=jnp.float32)
            + fc1b_ref[...], 0.0)
        out_ref[...] = (
            jnp.dot(g1, fc2w_ref[...], preferred_element_type=jnp.float32)
            + fc2b_ref[...])


_fin_kernel = pl.pallas_call(
    _fin_body,
    grid=(NRB,),
    in_specs=[
        pl.BlockSpec((RB, HIDDEN), lambda i: (i, 0)),
        pl.BlockSpec((NC, RB, HH), lambda i: (0, i, 0)),
        pl.BlockSpec((RB, 1), lambda i: (i, 0)),
        pl.BlockSpec((RB, 1), lambda i: (i, 0)),
        pl.BlockSpec((HIDDEN, 32), lambda i: (0, 0)),
        pl.BlockSpec((1, 32), lambda i: (0, 0)),
        pl.BlockSpec((32, 1), lambda i: (0, 0)),
        pl.BlockSpec((1, 1), lambda i: (0, 0)),
    ],
    out_specs=pl.BlockSpec((N_GRAPHS, 1), lambda i: (0, 0)),
    out_shape=jax.ShapeDtypeStruct((N_GRAPHS, 1), jnp.float32),
    scratch_shapes=[pltpu.VMEM((N_GRAPHS, HIDDEN), jnp.float32)],
)


def kernel(x, edge_index, lmax, batch,
           W1, b1, W2, b2, W3, b3, W4, b4,
           fc1_w, fc1_b, fc2_w, fc2_b):
    row = edge_index[0]
    col = edge_index[1]
    npad = E_PAD - N_EDGES
    pr = N_NODES + (jnp.arange(npad, dtype=jnp.int32) % (N_PAD - N_NODES))
    pc = jnp.arange(npad, dtype=jnp.int32) % N_NODES
    rw = jnp.concatenate([row, pr])
    cw = jnp.concatenate([col, pc])
    row3 = rw.reshape(NW, NJ, CH)
    r16 = rw.reshape(NS, NJ2, CH)
    c16 = cw.reshape(NS, NJ2, CH)
    row4 = jnp.concatenate([r16, r16], axis=0)
    col4 = jnp.concatenate([c16, c16 + N_NODES], axis=0)

    degp = _deg_kernel()(row3)
    degT = degp.T

    yt, base, cvec, dinv, diag = _tc1_kernel(
        x, degT, lmax.reshape(N_GRAPHS, 1), batch.reshape(N_NODES, 1),
        W1, b1.reshape(1, HIDDEN))
    for W, b in ((W2, b2), (W3, b3), (W4, b4)):
        p = _scat_kernel()(yt.reshape(2 * N_NODES, HH), row4, col4)
        yt, base = _tcl_kernel(base, p, cvec, dinv, diag, W,
                               b.reshape(1, HIDDEN))
    p = _scat_kernel()(yt.reshape(2 * N_NODES, HH), row4, col4)
    return _fin_kernel(base, p, cvec, batch.reshape(N_NODES, 1),
                       fc1_w, fc1_b.reshape(1, 32),
                       fc2_w, fc2_b.reshape(1, 1))
